# baseline (speedup 1.0000x reference)
import functools
import jax
import jax.numpy as jnp
from jax import lax
from jax.experimental import pallas as pl
from jax.experimental.pallas import tpu as pltpu
from jax.experimental.pallas import tpu_sc as plsc

B, NV, NE, D = 4, 10000, 160000, 128
NC, NS, L = 2, 16, 16
NW = NC * NS
EPW = NE // NW
KW = 80
NBLK = EPW // KW
KT = EPW - NBLK * KW
NIDX = 2 * EPW
R = 4
CP_ROWS = 1248
TL_ROWS = 640
TL_LAST = NV - 15 * TL_ROWS
SP = 40

_mesh = plsc.VectorSubcoreMesh(core_axis_name="c", subcore_axis_name="s")


@functools.partial(
    pl.kernel,
    out_type=jax.ShapeDtypeStruct((B, NV + NE, D), jnp.float32),
    mesh=_mesh,
    scratch_types=[
        pltpu.VMEM_SHARED((NV, D), jnp.float32),
        pltpu.VMEM((NIDX,), jnp.int32),
        pltpu.VMEM((KW, D), jnp.float32),
        pltpu.VMEM((KW, D), jnp.float32),
        pltpu.VMEM((KW, D), jnp.float32),
        pltpu.VMEM((KW, D), jnp.float32),
        pltpu.SemaphoreType.DMA,
        pltpu.SemaphoreType.DMA,
        pltpu.SemaphoreType.DMA,
        pltpu.SemaphoreType.DMA,
        pltpu.SemaphoreType.DMA,
        pltpu.SemaphoreType.DMA,
        pltpu.SemaphoreType.DMA,
        pltpu.SemaphoreType.DMA,
        pltpu.SemaphoreType.DMA,
    ],
)
def _unpool_kernel(vflat, ic, out,
                   table, idxall, r0, r1, r2, r3,
                   sg0, sg1, sg2, sg3, sw0, sw1, sw2, sw3, sc):
    rows = [r0, r1, r2, r3]
    semG = [sg0, sg1, sg2, sg3]
    semW = [sw0, sw1, sw2, sw3]

    cid = lax.axis_index("c")
    sid = lax.axis_index("s")
    wid = sid * NC + cid

    cb = wid // 8
    cr0 = (wid % 8) * CP_ROWS
    cp = pltpu.async_copy(vflat.at[pl.ds(cb * NV + cr0, CP_ROWS)],
                          out.at[cb, pl.ds(cr0, CP_ROWS)], sc)
    RREM = NV - 8 * CP_ROWS

    @pl.when(wid < B)
    def _():
        pltpu.async_copy(vflat.at[pl.ds(wid * NV + 8 * CP_ROWS, RREM)],
                         out.at[wid, pl.ds(8 * CP_ROWS, RREM)], sc)

    pltpu.sync_copy(ic.at[pl.ds(wid * NIDX, NIDX)], idxall)
    row0 = wid * EPW

    def i0_ref(w, n):
        return idxall.at[pl.ds(w * 2 * KW, n)]

    def i1_ref(w, n):
        return idxall.at[pl.ds(w * 2 * KW + n, n)]

    def dst(s, n):
        return rows[s] if n == KW else rows[s].at[pl.ds(0, n)]

    def fire_g1(s, w, n=KW):
        pltpu.async_copy(table.at[i0_ref(w, n)], dst(s, n), semG[s])

    def wait_g1(s, w, n=KW):
        pltpu.make_async_copy(table.at[i0_ref(w, n)], dst(s, n),
                              semG[s]).wait()

    def fire_g2(s, w, n=KW):
        pltpu.async_copy(table.at[i1_ref(w, n)], dst(s, n), semG[s], add=True)

    def wait_g2(s, w, n=KW):
        pltpu.make_async_copy(table.at[i1_ref(w, n)], dst(s, n),
                              semG[s]).wait()

    def wait_write(s, n=KW):
        pltpu.make_async_copy(dst(s, n), out.at[0, pl.ds(NV, n)],
                              semW[s]).wait()

    def fire_write(s, b, w, n=KW):
        pltpu.async_copy(dst(s, n),
                         out.at[b, pl.ds(NV + row0 + w * KW, n)], semW[s])

    for b in range(B):
        def scale_piece(r_off):
            stg = rows[0].at[pl.ds(0, SP)]
            pltpu.sync_copy(vflat.at[pl.ds(b * NV + r_off, SP)], stg)

            @plsc.parallel_loop(0, SP, unroll=2)
            def _(r):
                for j in range(D // L):
                    sl = pl.ds(j * L, L)
                    rows[0][r, sl] = rows[0][r, sl] * 0.5

            pltpu.sync_copy(stg, table.at[pl.ds(r_off, SP)])

        @pl.when(sid < NS - 1)
        def _():
            def pbody(i, carry):
                scale_piece(sid * TL_ROWS + i * SP)
                return carry

            lax.fori_loop(0, TL_ROWS // SP, pbody, 0)

        @pl.when(sid == NS - 1)
        def _():
            def pbody(i, carry):
                scale_piece(15 * TL_ROWS + i * SP)
                return carry

            lax.fori_loop(0, TL_LAST // SP, pbody, 0)

        plsc.subcore_barrier()

        fire_g1(0, 0)
        fire_g1(1, 1)

        def group_body(g, carry):
            for s in range(R):
                w = R * g + s
                ns = (s + 2) % R
                wait_g1(s, w)
                fire_g2(s, w)

                @pl.when(w >= 2)
                def _():
                    wait_write(ns)

                fire_g1(ns, w + 2)
                wait_g2(s, w)
                fire_write(s, b, w)
            return carry

        lax.fori_loop(0, (NBLK - 2) // R, group_body, 0)
        for w in (NBLK - 2, NBLK - 1, NBLK):
            s = w % R
            ns = (s + 2) % R
            n = KT if w == NBLK else KW
            wait_g1(s, w, n)
            fire_g2(s, w, n)
            if w + 2 <= NBLK:
                wait_write(ns)
                fire_g1(ns, w + 2, KT if w + 2 == NBLK else KW)
            wait_g2(s, w, n)
            fire_write(s, b, w, n)

        for w in (NBLK - 2, NBLK - 1, NBLK):
            wait_write(w % R, KT if w == NBLK else KW)
        wait_write((NBLK - 3) % R)
        plsc.subcore_barrier()

    cp.wait()

    @pl.when(wid < B)
    def _():
        pltpu.make_async_copy(vflat.at[pl.ds(wid * NV + 8 * CP_ROWS, RREM)],
                              out.at[wid, pl.ds(8 * CP_ROWS, RREM)], sc).wait()


def kernel(vertices, unpool_idx):
    vflat = vertices.reshape(B * NV, D)
    e = unpool_idx.reshape(NW, EPW, 2)
    full = e[:, :NBLK * KW, :].reshape(NW, NBLK, KW, 2)
    full = full.transpose(0, 1, 3, 2).reshape(NW, NBLK * 2 * KW)
    tail = e[:, NBLK * KW:, :].transpose(0, 2, 1).reshape(NW, 2 * KT)
    ic = jnp.concatenate([full, tail], axis=1).reshape(-1)
    return _unpool_kernel(vflat, ic)


# speedup vs baseline: 1.0028x; 1.0028x over previous
import functools
import jax
import jax.numpy as jnp
from jax import lax
from jax.experimental import pallas as pl
from jax.experimental.pallas import tpu as pltpu
from jax.experimental.pallas import tpu_sc as plsc

B, NV, NE, D = 4, 10000, 160000, 128
NC, NS, L = 2, 16, 16
NW = NC * NS
K = 64
NCHUNK = NE // K
CBASE, CREM = NCHUNK // NW, NCHUNK % NW
CP_ROWS = 1248
TL_ROWS = 640
TL_LAST = NV - 15 * TL_ROWS

_mesh = plsc.VectorSubcoreMesh(core_axis_name="c", subcore_axis_name="s")


@functools.partial(
    pl.kernel,
    out_type=jax.ShapeDtypeStruct((B, NV + NE, D), jnp.float32),
    mesh=_mesh,
    scratch_types=[
        pltpu.VMEM_SHARED((NV, D), jnp.float32),
        pltpu.VMEM((K,), jnp.int32),
        pltpu.VMEM((K,), jnp.int32),
        pltpu.VMEM((K,), jnp.int32),
        pltpu.VMEM((K,), jnp.int32),
        pltpu.VMEM((K, D), jnp.float32),
        pltpu.VMEM((K, D), jnp.float32),
        pltpu.VMEM((K, D), jnp.float32),
        pltpu.VMEM((K, D), jnp.float32),
        pltpu.SemaphoreType.DMA,
        pltpu.SemaphoreType.DMA,
        pltpu.SemaphoreType.DMA,
        pltpu.SemaphoreType.DMA,
        pltpu.SemaphoreType.DMA,
    ],
)
def _unpool_kernel(vflat, i0, i1, out,
                   table, ix0_0, ix0_1, ix1_0, ix1_1,
                   rA0, rA1, rB0, rB1,
                   sg0, sg1, sw0, sw1, sc):
    idx0 = [ix0_0, ix0_1]
    idx1 = [ix1_0, ix1_1]
    rowsA = [rA0, rA1]
    rowsB = [rB0, rB1]
    semG = [sg0, sg1]
    semW = [sw0, sw1]

    cid = lax.axis_index("c")
    sid = lax.axis_index("s")
    wid = sid * NC + cid

    cb = wid // 8
    cr0 = (wid % 8) * CP_ROWS
    cp = pltpu.async_copy(vflat.at[pl.ds(cb * NV + cr0, CP_ROWS)],
                          out.at[cb, pl.ds(cr0, CP_ROWS)], sc)
    RREM = NV - 8 * CP_ROWS

    @pl.when(wid < B)
    def _():
        pltpu.async_copy(vflat.at[pl.ds(wid * NV + 8 * CP_ROWS, RREM)],
                         out.at[wid, pl.ds(8 * CP_ROWS, RREM)], sc)

    cnt = CBASE + jnp.where(wid < CREM, 1, 0).astype(jnp.int32)
    lo = wid * CBASE + jnp.minimum(wid, CREM)

    def load_idx(p, c):
        pltpu.sync_copy(i0.at[pl.ds(c * K, K)], idx0[p])
        pltpu.sync_copy(i1.at[pl.ds(c * K, K)], idx1[p])

    def fire_gather(p):
        pltpu.async_copy(table.at[idx0[p]], rowsA[p], semG[p])
        pltpu.async_copy(table.at[idx1[p]], rowsB[p], semG[p])

    def wait_gather(p):
        pltpu.make_async_copy(table.at[idx0[p]], rowsA[p], semG[p]).wait()
        pltpu.make_async_copy(table.at[idx1[p]], rowsB[p], semG[p]).wait()

    def wait_write(p):
        pltpu.make_async_copy(rowsA[p], out.at[0, pl.ds(NV, K)], semW[p]).wait()

    for b in range(B):
        @pl.when(sid < NS - 1)
        def _():
            pltpu.sync_copy(vflat.at[pl.ds(b * NV + sid * TL_ROWS, TL_ROWS)],
                            table.at[pl.ds(sid * TL_ROWS, TL_ROWS)])

        @pl.when(sid == NS - 1)
        def _():
            pltpu.sync_copy(vflat.at[pl.ds(b * NV + 15 * TL_ROWS, TL_LAST)],
                            table.at[pl.ds(15 * TL_ROWS, TL_LAST)])

        plsc.subcore_barrier()

        load_idx(0, lo)
        fire_gather(0)

        def pair_body(g, carry):
            for p in (0, 1):
                t = 2 * g + p
                q = p ^ 1

                @pl.when(t < cnt)
                def _():
                    @pl.when(t + 1 < cnt)
                    def _():
                        @pl.when(t >= 1)
                        def _():
                            wait_write(q)

                        load_idx(q, lo + t + 1)
                        fire_gather(q)

                    wait_gather(p)

                    def row_body(r, rcarry):
                        for j in range(D // L):
                            sl = pl.ds(j * L, L)
                            rowsA[p][r, sl] = (rowsA[p][r, sl]
                                               + rowsB[p][r, sl]) * 0.5
                        return rcarry

                    lax.fori_loop(0, K, row_body, 0)
                    pltpu.async_copy(
                        rowsA[p], out.at[b, pl.ds(NV + (lo + t) * K, K)],
                        semW[p])
            return carry

        lax.fori_loop(0, (CBASE + 2) // 2, pair_body, 0)
        wait_write(0)
        wait_write(1)
        plsc.subcore_barrier()

    cp.wait()

    @pl.when(wid < B)
    def _():
        pltpu.make_async_copy(vflat.at[pl.ds(wid * NV + 8 * CP_ROWS, RREM)],
                              out.at[wid, pl.ds(8 * CP_ROWS, RREM)], sc).wait()


def kernel(vertices, unpool_idx):
    vflat = vertices.reshape(B * NV, D)
    i0 = unpool_idx[:, 0]
    i1 = unpool_idx[:, 1]
    return _unpool_kernel(vflat, i0, i1)
